# baseline (speedup 1.0000x reference)
import functools
import math

import jax
import jax.numpy as jnp
from jax import lax
from jax.experimental import pallas as pl
from jax.experimental.pallas import tpu as pltpu
from jax.experimental.pallas import tpu_sc as plsc

VOCAB = 1_000_000
D = 64
SCALE = math.sqrt(D)

_info = plsc.get_sparse_core_info()
NC = _info.num_cores
NS = _info.num_subcores
NW = NC * NS
L = _info.num_lanes

CHUNK = 128
NBUF = 2


def _build(S, T):
    B = S * T
    nrows = B // CHUNK
    spt = S // CHUNK
    per_w = nrows // NW
    nouter = per_w // NBUF

    mesh = plsc.VectorSubcoreMesh(core_axis_name="c", subcore_axis_name="s")

    @functools.partial(
        pl.kernel,
        mesh=mesh,
        compiler_params=pltpu.CompilerParams(
            use_tc_tiling_on_sc=False, needs_layout_passes=False),
        out_type=jax.ShapeDtypeStruct((T * D, S), jnp.float32),
        scratch_types=[
            pltpu.VMEM((per_w, CHUNK), jnp.int32),
            [pltpu.VMEM((CHUNK, D), jnp.float32) for _ in range(NBUF)],
            [pltpu.VMEM((D, CHUNK), jnp.float32) for _ in range(NBUF)],
            [pltpu.SemaphoreType.DMA for _ in range(NBUF)],
            [pltpu.SemaphoreType.DMA for _ in range(NBUF)],
        ],
    )
    def emb(tok_hbm, table_hbm, out_hbm, idx_v, bin_, bout, gsem, osem):
        wid = lax.axis_index("s") * NC + lax.axis_index("c")
        pltpu.sync_copy(tok_hbm.at[pl.ds(wid * per_w, per_w)], idx_v)
        g0 = wid * per_w

        def gather(j, b):
            pltpu.async_copy(table_hbm.at[idx_v.at[j]], bin_[b], gsem[b])

        def put(j, b):
            g = g0 + j
            t = g // spt
            s0 = (g % spt) * CHUNK
            pltpu.async_copy(
                bout[b], out_hbm.at[pl.ds(t * D, D), pl.ds(s0, CHUNK)], osem[b])

        def put_wait(b):
            pltpu.make_async_copy(
                bout[b], out_hbm.at[pl.ds(0, D), pl.ds(0, CHUNK)], osem[b]).wait()

        for b in range(NBUF):
            gather(b, b)

        lanes = lax.iota(jnp.int32, L)

        def outer(jj, _):
            for b in range(NBUF):
                j = jj * NBUF + b
                pltpu.make_async_copy(table_hbm.at[idx_v.at[j]], bin_[b],
                                      gsem[b]).wait()

                @pl.when(jj > 0)
                def _():
                    put_wait(b)

                def tr_body(d, _):
                    dcol = jnp.full((L,), d, jnp.int32)
                    for sv in range(CHUNK // L):
                        vals = plsc.load_gather(
                            bin_[b], [sv * L + lanes, dcol]) * SCALE
                        bout[b][d, pl.ds(sv * L, L)] = vals
                    return ()

                lax.fori_loop(0, D, tr_body, (), unroll=4)
                put(j, b)

                @pl.when(jj < nouter - 1)
                def _():
                    gather(j + NBUF, b)
            return ()

        lax.fori_loop(0, nouter, outer, ())
        for b in range(NBUF):
            put_wait(b)

    return emb


def kernel(tokens, table):
    S, T = tokens.shape
    tok2d = tokens.T.reshape((S * T) // CHUNK, CHUNK).astype(jnp.int32)
    out = _build(S, T)(tok2d, table)
    return out.reshape(T, D, S).transpose(2, 0, 1)


# speedup vs baseline: 1.5855x; 1.5855x over previous
import functools
import math

import jax
import jax.numpy as jnp
from jax import lax
from jax.experimental import pallas as pl
from jax.experimental.pallas import tpu as pltpu
from jax.experimental.pallas import tpu_sc as plsc

VOCAB = 1_000_000
D = 64
SCALE = math.sqrt(D)

_info = plsc.get_sparse_core_info()
NC = _info.num_cores
NS = _info.num_subcores
NW = NC * NS
L = _info.num_lanes

CHUNK = 128
NBUF = 2


def _build(S, T):
    B = S * T
    nrows = B // CHUNK
    spt = S // CHUNK
    per_w = nrows // NW
    nouter = per_w // NBUF

    mesh = plsc.VectorSubcoreMesh(core_axis_name="c", subcore_axis_name="s")

    @functools.partial(
        pl.kernel,
        mesh=mesh,
        compiler_params=pltpu.CompilerParams(
            use_tc_tiling_on_sc=False, needs_layout_passes=False),
        out_type=jax.ShapeDtypeStruct((T * D, S), jnp.float32),
        scratch_types=[
            pltpu.VMEM((per_w, CHUNK), jnp.int32),
            [pltpu.VMEM((CHUNK, D), jnp.float32) for _ in range(NBUF)],
            [pltpu.VMEM((D, CHUNK + 1), jnp.float32) for _ in range(NBUF)],
            [pltpu.SemaphoreType.DMA for _ in range(NBUF)],
            [pltpu.SemaphoreType.DMA for _ in range(NBUF)],
        ],
    )
    def emb(tok_hbm, table_hbm, out_hbm, idx_v, bin_, bout, gsem, osem):
        wid = lax.axis_index("s") * NC + lax.axis_index("c")
        pltpu.sync_copy(tok_hbm.at[pl.ds(wid * per_w, per_w)], idx_v)
        g0 = wid * per_w

        def gather(j, b):
            pltpu.async_copy(table_hbm.at[idx_v.at[j]], bin_[b], gsem[b])

        def put(j, b):
            g = g0 + j
            t = g // spt
            s0 = (g % spt) * CHUNK
            pltpu.async_copy(
                bout[b].at[:, pl.ds(0, CHUNK)],
                out_hbm.at[pl.ds(t * D, D), pl.ds(s0, CHUNK)], osem[b])

        def put_wait(b):
            pltpu.make_async_copy(
                bout[b].at[:, pl.ds(0, CHUNK)],
                out_hbm.at[pl.ds(0, D), pl.ds(0, CHUNK)], osem[b]).wait()

        for b in range(NBUF):
            gather(b, b)

        lanes = lax.iota(jnp.int32, L)

        def outer(jj, _):
            for b in range(NBUF):
                j = jj * NBUF + b
                pltpu.make_async_copy(table_hbm.at[idx_v.at[j]], bin_[b],
                                      gsem[b]).wait()

                @pl.when(jj > 0)
                def _():
                    put_wait(b)

                def tr_body(s, _):
                    scol = jnp.full((L,), s, jnp.int32)
                    for dv in range(D // L):
                        row = bin_[b][s, pl.ds(dv * L, L)] * SCALE
                        plsc.store_scatter(bout[b], [dv * L + lanes, scol], row)
                    return ()

                lax.fori_loop(0, CHUNK, tr_body, (), unroll=4)
                put(j, b)

                @pl.when(jj < nouter - 1)
                def _():
                    gather(j + NBUF, b)
            return ()

        lax.fori_loop(0, nouter, outer, ())
        for b in range(NBUF):
            put_wait(b)

    return emb


def kernel(tokens, table):
    S, T = tokens.shape
    tok2d = tokens.T.reshape((S * T) // CHUNK, CHUNK).astype(jnp.int32)
    out = _build(S, T)(tok2d, table)
    return out.reshape(T, D, S).transpose(2, 0, 1)
